# Initial kernel scaffold; baseline (speedup 1.0000x reference)
import jax
import jax.numpy as jnp
from jax.experimental import pallas as pl


def kernel(x, edge_index, W1, b1, W2, b2, Wfc, bfc):
    raise NotImplementedError("write your pallas kernel here")


# speedup vs baseline: 30.7517x; 30.7517x over previous
import functools

import jax
import jax.numpy as jnp
from jax import lax
from jax.experimental import pallas as pl
from jax.experimental.pallas import tpu as pltpu
from jax.experimental.pallas import tpu_sc as plsc

N_NODES = 10000
N_EDGES = 320000
D = 128

NC = 2
NS = 16
NW = NC * NS

NP = 10240
ROWS_PER_TILE = NP // NS
CHUNK = 128
GK = 16
G = 5
CHUNKS = G * GK
EW = CHUNKS * CHUNK
E_PAD = NW * EW

NB = 5
BR = NP // NB


def _sc_mesh():
    return plsc.VectorSubcoreMesh(core_axis_name="c", subcore_axis_name="s",
                                  num_cores=NC, num_subcores=NS)


@functools.cache
def _build_deg_kernel():
    return functools.partial(
        pl.kernel,
        out_type=jax.ShapeDtypeStruct((NC, NP), jnp.float32),
        mesh=_sc_mesh(),
        scratch_types=[
            pltpu.VMEM((CHUNKS, CHUNK), jnp.int32),
            pltpu.VMEM((CHUNK,), jnp.float32),
            pltpu.VMEM_SHARED((NP,), jnp.float32),
            pltpu.SemaphoreType.DMA,
        ],
    )(_deg_body)


def _deg_body(dst_hbm, ones_hbm, zeros1_hbm, out_hbm, idx_v, ones_v, acc_sh, sem):
    c = lax.axis_index("c")
    s = lax.axis_index("s")
    wid = s * NC + c
    pltpu.sync_copy(ones_hbm, ones_v)
    pltpu.sync_copy(zeros1_hbm.at[pl.ds(s * ROWS_PER_TILE, ROWS_PER_TILE)],
                    acc_sh.at[pl.ds(s * ROWS_PER_TILE, ROWS_PER_TILE)])
    pltpu.sync_copy(dst_hbm.at[wid], idx_v)
    plsc.subcore_barrier()

    AHEAD = 8

    def fire(j):
        pltpu.async_copy(ones_v, acc_sh.at[idx_v.at[j]], sem, add=True)

    def drain(j, carry):
        pltpu.make_async_copy(ones_v, acc_sh.at[idx_v.at[0]], sem).wait()
        return carry

    for j in range(AHEAD):
        fire(j)

    def body(j, carry):
        pltpu.make_async_copy(ones_v, acc_sh.at[idx_v.at[0]], sem).wait()
        fire(j + AHEAD)
        return carry

    lax.fori_loop(0, CHUNKS - AHEAD, body, 0)
    lax.fori_loop(0, AHEAD, drain, 0)
    plsc.subcore_barrier()
    pltpu.sync_copy(acc_sh.at[pl.ds(s * ROWS_PER_TILE, ROWS_PER_TILE)],
                    out_hbm.at[c, pl.ds(s * ROWS_PER_TILE, ROWS_PER_TILE)])


@functools.cache
def _build_edge_sum_kernel():
    return functools.partial(
        pl.kernel,
        out_type=jax.ShapeDtypeStruct((NC, NP, D), jnp.float32),
        mesh=_sc_mesh(),
        scratch_types=[
            pltpu.VMEM((2, GK, CHUNK), jnp.int32),
            pltpu.VMEM((CHUNK, D), jnp.float32),
            pltpu.VMEM((CHUNK, D), jnp.float32),
            pltpu.VMEM_SHARED((NP, D), jnp.float32),
            pltpu.SemaphoreType.DMA,
            pltpu.SemaphoreType.DMA,
            pltpu.SemaphoreType.DMA,
        ],
    )(_edge_sum_body)


def _edge_sum_body(y_hbm, src_hbm, dst_hbm, zeros2_hbm, out_hbm,
                   idxw, rows0, rows1, acc_sh, g0sem, g1sem, ssem):
    c = lax.axis_index("c")
    s = lax.axis_index("s")
    wid = s * NC + c
    pltpu.sync_copy(zeros2_hbm,
                    acc_sh.at[pl.ds(s * ROWS_PER_TILE, ROWS_PER_TILE)])
    plsc.subcore_barrier()

    bufs = ((rows0, g0sem), (rows1, g1sem))

    def group(g, carry):
        pltpu.sync_copy(src_hbm.at[wid, g], idxw.at[0])
        pltpu.sync_copy(dst_hbm.at[wid, g], idxw.at[1])
        pltpu.async_copy(y_hbm.at[idxw.at[0, 0]], rows0, g0sem)
        for k in range(GK):
            rbuf, rsem = bufs[k % 2]
            if k > 0:
                pbuf = bufs[(k - 1) % 2][0]
                pltpu.make_async_copy(pbuf, acc_sh.at[idxw.at[1, k - 1]],
                                      ssem).wait()
            if k + 1 < GK:
                nbuf, nsem = bufs[(k + 1) % 2]
                pltpu.async_copy(y_hbm.at[idxw.at[0, k + 1]], nbuf, nsem)
            pltpu.make_async_copy(y_hbm.at[idxw.at[0, k]], rbuf, rsem).wait()
            pltpu.async_copy(rbuf, acc_sh.at[idxw.at[1, k]], ssem, add=True)
        last = bufs[(GK - 1) % 2][0]
        pltpu.make_async_copy(last, acc_sh.at[idxw.at[1, GK - 1]],
                              ssem).wait()
        return carry

    lax.fori_loop(0, G, group, 0)

    plsc.subcore_barrier()
    pltpu.sync_copy(acc_sh.at[pl.ds(s * ROWS_PER_TILE, ROWS_PER_TILE)],
                    out_hbm.at[c, pl.ds(s * ROWS_PER_TILE, ROWS_PER_TILE)])


def _tc_a1_body(x_ref, w1_ref, xw_ref):
    xw_ref[...] = jnp.dot(x_ref[...], w1_ref[...],
                          preferred_element_type=jnp.float32)


def _tc_a2_body(xw_ref, p0_ref, p1_ref, y_ref, dinv_ref):
    deg = p0_ref[0] + p1_ref[0] + 1.0
    dinv = lax.rsqrt(jnp.maximum(deg, 1.0))
    y_ref[...] = xw_ref[...] * dinv
    dinv_ref[...] = dinv


def _tc_b_body(a0_ref, a1_ref, y1_ref, dinv_ref, b1_ref, w2_ref, y2_ref):
    dinv = dinv_ref[...]
    acc = a0_ref[0] + a1_ref[0] + y1_ref[...]
    h1 = jnp.maximum(acc * dinv + b1_ref[...], 0.0)
    y2_ref[...] = jnp.dot(h1, w2_ref[...],
                          preferred_element_type=jnp.float32) * dinv


def _tc_c_body(a0_ref, a1_ref, y2_ref, dinv_ref, b2_ref, wfc_ref, bfc_ref,
               out_ref, acc_scr):
    i = pl.program_id(0)
    acc = a0_ref[0] + a1_ref[0] + y2_ref[...]
    h2 = jnp.maximum(acc * dinv_ref[...] + b2_ref[...], 0.0)
    row = lax.broadcasted_iota(jnp.int32, (BR, D), 0) + i * BR
    h2 = jnp.where(row < N_NODES, h2, 0.0)
    psum = jnp.sum(h2, axis=0, keepdims=True)

    @pl.when(i == 0)
    def _():
        acc_scr[...] = psum

    @pl.when(i > 0)
    def _():
        acc_scr[...] = acc_scr[...] + psum

    @pl.when(i == NB - 1)
    def _():
        pooled = acc_scr[...] / float(N_NODES)
        tot = jnp.sum(pooled * wfc_ref[...], axis=1, keepdims=True)
        out_ref[...] = jax.nn.sigmoid(jnp.broadcast_to(tot, (1, D))
                                      + bfc_ref[...])


def _row_spec(shape):
    return pl.BlockSpec(shape, lambda i: (i, 0))


def _full_spec(shape):
    return pl.BlockSpec(shape, lambda i: (0, 0))


def _part_spec(c):
    if c == 0:
        return pl.BlockSpec((1, BR, D), lambda i: (0, i, 0))
    return pl.BlockSpec((1, BR, D), lambda i: (1, i, 0))


def _deg_spec(c):
    if c == 0:
        return pl.BlockSpec((1, BR, 1), lambda i: (0, i, 0))
    return pl.BlockSpec((1, BR, 1), lambda i: (1, i, 0))


_tc_a1 = pl.pallas_call(
    _tc_a1_body,
    grid=(NB,),
    in_specs=[_row_spec((BR, D)), _full_spec((D, D))],
    out_specs=_row_spec((BR, D)),
    out_shape=jax.ShapeDtypeStruct((NP, D), jnp.float32),
)

_tc_a2 = pl.pallas_call(
    _tc_a2_body,
    grid=(NB,),
    in_specs=[_row_spec((BR, D)), _deg_spec(0), _deg_spec(1)],
    out_specs=[_row_spec((BR, D)), _row_spec((BR, 1))],
    out_shape=[jax.ShapeDtypeStruct((NP, D), jnp.float32),
               jax.ShapeDtypeStruct((NP, 1), jnp.float32)],
)

_tc_b = pl.pallas_call(
    _tc_b_body,
    grid=(NB,),
    in_specs=[_part_spec(0), _part_spec(1), _row_spec((BR, D)),
              _row_spec((BR, 1)), _full_spec((1, D)), _full_spec((D, D))],
    out_specs=_row_spec((BR, D)),
    out_shape=jax.ShapeDtypeStruct((NP, D), jnp.float32),
)

_tc_c = pl.pallas_call(
    _tc_c_body,
    grid=(NB,),
    in_specs=[_part_spec(0), _part_spec(1), _row_spec((BR, D)),
              _row_spec((BR, 1)), _full_spec((1, D)), _full_spec((1, D)),
              _full_spec((1, D))],
    out_specs=_full_spec((1, D)),
    out_shape=jax.ShapeDtypeStruct((1, D), jnp.float32),
    scratch_shapes=[pltpu.VMEM((1, D), jnp.float32)],
)


def kernel(x, edge_index, W1, b1, W2, b2, Wfc, bfc):
    src = edge_index[0].astype(jnp.int32)
    dst = edge_index[1].astype(jnp.int32)

    n_dummy = E_PAD - N_EDGES
    pad_rows = N_NODES + (jnp.arange(n_dummy, dtype=jnp.int32) % (NP - N_NODES))
    src_flat = jnp.concatenate([src, pad_rows])
    dst_flat = jnp.concatenate([dst, pad_rows])
    src_p = src_flat.reshape(NW, G, GK, CHUNK)
    dst_p = dst_flat.reshape(NW, G, GK, CHUNK)
    dst_deg = dst_flat.reshape(NW, CHUNKS, CHUNK)

    x_p = jnp.pad(x, ((0, NP - N_NODES), (0, 0)))
    ones = jnp.ones((CHUNK,), jnp.float32)
    zeros1 = jnp.zeros((NP,), jnp.float32)
    zeros2 = jnp.zeros((ROWS_PER_TILE, D), jnp.float32)

    xw1 = _tc_a1(x_p, W1)
    degs = _build_deg_kernel()(dst_deg, ones, zeros1)
    degs3 = degs.reshape(NC, NP, 1)

    y1, dinv = _tc_a2(xw1, degs3, degs3)

    edge_sum = _build_edge_sum_kernel()
    acc1 = edge_sum(y1, src_p, dst_p, zeros2)
    y2 = _tc_b(acc1, acc1, y1, dinv, b1.reshape(1, D), W2)

    acc2 = edge_sum(y2, src_p, dst_p, zeros2)
    out = _tc_c(acc2, acc2, y2, dinv, b2.reshape(1, D),
                Wfc.reshape(1, D), jnp.broadcast_to(bfc.reshape(1, 1), (1, D)))
    return out[0, :1]
